# baseline (speedup 1.0000x reference)
import jax
import jax.numpy as jnp
from jax import lax
from jax.experimental import pallas as pl
from jax.experimental.pallas import tpu as pltpu
from jax.experimental.pallas import tpu_sc as plsc

_FEAT_DIM = 128
_ALPHA = 0.001
_BETA = 0.5
_EPS = 1e-8
_T = 8192
_HALF = 32
_K = 6
_NBLK = 64
_BLK = 128


def _loss1_body(nor_ref, cen_ref, out_ref, acc_ref):
    i = pl.program_id(0)

    @pl.when(i == 0)
    def _init():
        acc_ref[0, 0] = jnp.float32(0.0)

    x = nor_ref[0]
    c = cen_ref[0]
    cinv = lax.rsqrt(jnp.sum(c * c))
    xb = x.astype(jnp.bfloat16)
    xcat = jnp.concatenate([xb, xb * xb], axis=1)
    cb = c.astype(jnp.bfloat16)
    z = jnp.zeros((_FEAT_DIM,), jnp.bfloat16)
    o = jnp.ones((_FEAT_DIM,), jnp.bfloat16)
    w = jnp.stack([jnp.concatenate([cb, z]), jnp.concatenate([z, o])], axis=1)
    p = lax.dot_general(xcat, w, (((1,), (0,)), ((), ())),
                        preferred_element_type=jnp.float32)
    pt = p.T
    dots = pt[0:1, :]
    n2 = pt[1:2, :]
    r = jnp.minimum(lax.rsqrt(n2) * cinv, jnp.float32(1.0 / _EPS))
    acc_ref[0, 0] += jnp.sum(dots * r)

    @pl.when(i == pl.num_programs(0) - 1)
    def _fin():
        mean_cos = acc_ref[0, 0] / jnp.float32(_HALF * _T)
        out_ref[0, 0] = _ALPHA * (1.0 - mean_cos)


def _iota16():
    return lax.broadcasted_iota(jnp.int32, (16,), 0)


def _bcast_max(v):
    it = _iota16()
    for sh in (1, 2, 4, 8):
        v = jnp.maximum(v, jnp.take(v, it ^ sh))
    return v


def _bcast_min(v):
    it = _iota16()
    for sh in (1, 2, 4, 8):
        v = jnp.minimum(v, jnp.take(v, it ^ sh))
    return v


def _bcast_sum(v):
    it = _iota16()
    for sh in (1, 2, 4, 8):
        v = v + jnp.take(v, it ^ sh)
    return v


def _nrsqrt(x):
    xh = x * 0.5
    i = lax.bitcast_convert_type(x, jnp.int32)
    i = jnp.int32(0x5F3759DF) - lax.shift_right_logical(i, 1)
    y = lax.bitcast_convert_type(i, jnp.float32)
    for _ in range(4):
        y = y * (1.5 - xh * y * y)
    return y


def _sc_triplet(score_hbm, feat_hbm, cen_hbm, out_hbm, pvrows_hbm,
                srow_a, bmax, bmin, cvec_r, idxg_r, rows_r,
                pref, tmp_r, sem):
    cid = lax.axis_index("c")
    sid = lax.axis_index("s")
    wid = cid * 16 + sid
    video = _HALF + wid
    iota16 = _iota16()
    ninf = jnp.float32(-jnp.inf)
    pinf = jnp.float32(jnp.inf)
    ibig = jnp.int32(1 << 30)

    pltpu.sync_copy(score_hbm.at[video], srow_a)
    pltpu.sync_copy(cen_hbm, cvec_r)

    def _blk(j, carry):
        base = j * _BLK
        mx = srow_a[pl.ds(base, 16)]
        mn = mx
        for k in range(1, 8):
            v = srow_a[pl.ds(base + k * 16, 16)]
            mx = jnp.maximum(mx, v)
            mn = jnp.minimum(mn, v)
        bmxv = _bcast_max(mx)
        bmnv = _bcast_min(mn)
        cb = (j // 16) * 16
        lane = j - cb
        bmax[pl.ds(cb, 16)] = jnp.where(iota16 == lane, bmxv, bmax[pl.ds(cb, 16)])
        bmin[pl.ds(cb, 16)] = jnp.where(iota16 == lane, bmnv, bmin[pl.ds(cb, 16)])
        return carry

    lax.fori_loop(0, _NBLK, _blk, 0, unroll=4)

    def _find_first(ref, base, valv):
        t = jnp.full((16,), 1 << 30, jnp.int32)
        for k in range(8):
            v = ref[pl.ds(base + k * 16, 16)]
            cand = jnp.where(v == valv, iota16 + (base + k * 16), ibig)
            t = jnp.minimum(t, cand)
        return _bcast_min(t)[0]

    def _block_stat(ref, base, ismax):
        a = ref[pl.ds(base, 16)]
        for k in range(1, 8):
            v = ref[pl.ds(base + k * 16, 16)]
            a = jnp.maximum(a, v) if ismax else jnp.minimum(a, v)
        return _bcast_max(a) if ismax else _bcast_min(a)

    def _rmw(ref, pos, valv):
        cb = (pos // 16) * 16
        lane = pos - cb
        ch = ref[pl.ds(cb, 16)]
        ref[pl.ds(cb, 16)] = jnp.where(iota16 == lane, valv, ch)

    idxg_vec = jnp.full((16,), video * _T, jnp.int32)

    saved = []
    for slot in range(12):
        is_top = slot >= _K
        if slot == _K:
            for tprev, gprev in saved:
                _rmw(srow_a, tprev, gprev)
        sref = srow_a
        bref = bmax if is_top else bmin
        ch0 = bref[pl.ds(0, 16)]
        ch1 = bref[pl.ds(16, 16)]
        ch2 = bref[pl.ds(32, 16)]
        ch3 = bref[pl.ds(48, 16)]
        if is_top:
            gmv = _bcast_max(jnp.maximum(jnp.maximum(ch0, ch1),
                                         jnp.maximum(ch2, ch3)))
        else:
            gmv = _bcast_min(jnp.minimum(jnp.minimum(ch0, ch1),
                                         jnp.minimum(ch2, ch3)))
        jcand = jnp.full((16,), 1 << 30, jnp.int32)
        for q, chq in enumerate((ch0, ch1, ch2, ch3)):
            jcand = jnp.minimum(jcand,
                                jnp.where(chq == gmv, iota16 + q * 16, ibig))
        jstar = _bcast_min(jcand)[0]
        base = jstar * _BLK
        tstar = _find_first(sref, base, gmv)
        maskv = jnp.full((16,), ninf if is_top else pinf, jnp.float32)
        _rmw(sref, tstar, maskv)
        _rmw(bref, jstar, _block_stat(sref, base, is_top))
        if not is_top:
            saved.append((tstar, gmv))
        idxg_vec = jnp.where(iota16 == slot, video * _T + tstar, idxg_vec)

    idxg_r[...] = idxg_vec
    pltpu.async_copy(feat_hbm.at[idxg_r], rows_r, sem).wait()

    cacc = jnp.zeros((16,), jnp.float32)
    for k in range(8):
        cv = cvec_r[pl.ds(k * 16, 16)]
        cacc = cacc + cv * cv
    cn2v = _bcast_sum(cacc)

    dv = jnp.zeros((16,), jnp.float32)
    nv = jnp.ones((16,), jnp.float32)
    for r in range(12):
        ad = jnp.zeros((16,), jnp.float32)
        an = jnp.zeros((16,), jnp.float32)
        for k in range(8):
            v = rows_r[r, pl.ds(k * 16, 16)]
            cv = cvec_r[pl.ds(k * 16, 16)]
            ad = ad + v * cv
            an = an + v * v
        dv = jnp.where(iota16 == r, _bcast_sum(ad), dv)
        nv = jnp.where(iota16 == r, _bcast_sum(an), nv)

    mu = jnp.minimum(_nrsqrt(nv) * _nrsqrt(cn2v), jnp.float32(1.0 / _EPS))
    d = (1.0 - dv * mu) * 0.5
    dn = jnp.take(d, jnp.minimum(iota16 + _K, jnp.int32(15)))
    z = jnp.maximum(d - dn + 1.0, 0.0)
    z = jnp.where(iota16 < _K, z, jnp.float32(0.0))
    pvv = _bcast_sum(z)

    pref[...] = jnp.where(iota16 == 0, pvv, jnp.float32(0.0))
    pltpu.sync_copy(pref, pvrows_hbm.at[wid])
    plsc.subcore_barrier()

    @pl.when(sid == 0)
    def _agg():
        pltpu.sync_copy(pvrows_hbm.at[pl.ds(cid * 16, 16)], tmp_r)
        tot = jnp.zeros((16,), jnp.float32)
        for v2 in range(16):
            tot = tot + tmp_r[v2, pl.ds(0, 16)]
        l2pv = _bcast_sum(tot) * jnp.float32(_BETA / (_HALF * _K))
        pref[...] = jnp.where(iota16 == 0, l2pv, jnp.float32(0.0))
        pltpu.sync_copy(pref, out_hbm.at[cid])


def kernel(feat, score, centers):
    score2 = score.reshape(2 * _HALF, _T)
    feat2 = feat.reshape(2 * _HALF * _T, _FEAT_DIM)
    featg = feat.reshape(4 * _HALF, _T // 2, _FEAT_DIM)
    cen2 = centers.reshape(1, _FEAT_DIM)

    l1 = pl.pallas_call(
        _loss1_body,
        grid=(2 * _HALF,),
        in_specs=[
            pl.BlockSpec((1, _T // 2, _FEAT_DIM), lambda i: (i, 0, 0)),
            pl.BlockSpec((1, _FEAT_DIM), lambda i: (0, 0)),
        ],
        out_specs=pl.BlockSpec((1, 1), lambda i: (0, 0),
                               memory_space=pltpu.SMEM),
        out_shape=jax.ShapeDtypeStruct((1, 1), jnp.float32),
        scratch_shapes=[pltpu.SMEM((1, 1), jnp.float32)],
    )(featg, cen2)

    mesh = plsc.VectorSubcoreMesh(core_axis_name="c", subcore_axis_name="s")
    l2, _ = pl.kernel(
        _sc_triplet,
        out_type=[jax.ShapeDtypeStruct((2, 16), jnp.float32),
                  jax.ShapeDtypeStruct((2 * _HALF // 2, 16), jnp.float32)],
        mesh=mesh,
        scratch_types=[
            pltpu.VMEM((_T,), jnp.float32),
            pltpu.VMEM((_NBLK,), jnp.float32),
            pltpu.VMEM((_NBLK,), jnp.float32),
            pltpu.VMEM((_FEAT_DIM,), jnp.float32),
            pltpu.VMEM((16,), jnp.int32),
            pltpu.VMEM((16, _FEAT_DIM), jnp.float32),
            pltpu.VMEM((16,), jnp.float32),
            pltpu.VMEM((16, 16), jnp.float32),
            pltpu.SemaphoreType.DMA,
        ],
    )(score2, feat2, centers)

    return l1[0, 0] + l2[0, 0] + l2[1, 0]


# speedup vs baseline: 1.2647x; 1.2647x over previous
import jax
import jax.numpy as jnp
from jax import lax
from jax.experimental import pallas as pl
from jax.experimental.pallas import tpu as pltpu
from jax.experimental.pallas import tpu_sc as plsc

_FEAT_DIM = 128
_ALPHA = 0.001
_BETA = 0.5
_EPS = 1e-8
_T = 8192
_HALF = 32
_K = 6
_NBLK = 64
_BLK = 128


def _loss1_body(nor_ref, cen_ref, out_ref, acc_ref):
    i = pl.program_id(0)

    @pl.when(i == 0)
    def _init():
        acc_ref[0, 0] = jnp.float32(0.0)

    x = nor_ref[0]
    c = cen_ref[0]
    cinv = lax.rsqrt(jnp.sum(c * c))
    xb = x.astype(jnp.bfloat16)
    xcat = jnp.concatenate([xb, xb * xb], axis=1)
    cb = c.astype(jnp.bfloat16)
    z = jnp.zeros((_FEAT_DIM,), jnp.bfloat16)
    o = jnp.ones((_FEAT_DIM,), jnp.bfloat16)
    w = jnp.stack([jnp.concatenate([cb, z]), jnp.concatenate([z, o])], axis=1)
    p = lax.dot_general(xcat, w, (((1,), (0,)), ((), ())),
                        preferred_element_type=jnp.float32)
    pt = p.T
    dots = pt[0:1, :]
    n2 = pt[1:2, :]
    r = jnp.minimum(lax.rsqrt(n2) * cinv, jnp.float32(1.0 / _EPS))
    acc_ref[0, 0] += jnp.sum(dots * r)

    @pl.when(i == pl.num_programs(0) - 1)
    def _fin():
        mean_cos = acc_ref[0, 0] / jnp.float32(_HALF * _T)
        out_ref[0, 0] = _ALPHA * (1.0 - mean_cos)


def _iota16():
    return lax.broadcasted_iota(jnp.int32, (16,), 0)


def _bcast_max(v):
    it = _iota16()
    for sh in (1, 2, 4, 8):
        v = jnp.maximum(v, jnp.take(v, it ^ sh))
    return v


def _bcast_min(v):
    it = _iota16()
    for sh in (1, 2, 4, 8):
        v = jnp.minimum(v, jnp.take(v, it ^ sh))
    return v


def _bcast_sum(v):
    it = _iota16()
    for sh in (1, 2, 4, 8):
        v = v + jnp.take(v, it ^ sh)
    return v


def _nrsqrt(x):
    xh = x * 0.5
    i = lax.bitcast_convert_type(x, jnp.int32)
    i = jnp.int32(0x5F3759DF) - lax.shift_right_logical(i, 1)
    y = lax.bitcast_convert_type(i, jnp.float32)
    for _ in range(4):
        y = y * (1.5 - xh * y * y)
    return y


def _sc_triplet(score_hbm, feat_hbm, cen_hbm, out_hbm, pvrows_hbm,
                srow_a, bmax, bmin, cvec_r, idxg_r, rows_r,
                pref, tmp_r, sem):
    cid = lax.axis_index("c")
    sid = lax.axis_index("s")
    wid = cid * 16 + sid
    video = _HALF + wid
    iota16 = _iota16()
    ninf = jnp.float32(-jnp.inf)
    pinf = jnp.float32(jnp.inf)
    ibig = jnp.int32(1 << 30)

    pltpu.sync_copy(score_hbm.at[video], srow_a)
    pltpu.sync_copy(cen_hbm, cvec_r)

    def _blk(j, carry):
        base = j * _BLK
        mx = srow_a[pl.ds(base, 16)]
        mn = mx
        for k in range(1, 8):
            v = srow_a[pl.ds(base + k * 16, 16)]
            mx = jnp.maximum(mx, v)
            mn = jnp.minimum(mn, v)
        bmxv = _bcast_max(mx)
        bmnv = _bcast_min(mn)
        cb = (j // 16) * 16
        lane = j - cb
        bmax[pl.ds(cb, 16)] = jnp.where(iota16 == lane, bmxv, bmax[pl.ds(cb, 16)])
        bmin[pl.ds(cb, 16)] = jnp.where(iota16 == lane, bmnv, bmin[pl.ds(cb, 16)])
        return carry

    lax.fori_loop(0, _NBLK, _blk, 0, unroll=4)

    def _find_first(ref, base, valv):
        t = jnp.full((16,), 1 << 30, jnp.int32)
        for k in range(8):
            v = ref[pl.ds(base + k * 16, 16)]
            cand = jnp.where(v == valv, iota16 + (base + k * 16), ibig)
            t = jnp.minimum(t, cand)
        return _bcast_min(t)[0]

    def _block_stat(ref, base, ismax):
        a = ref[pl.ds(base, 16)]
        for k in range(1, 8):
            v = ref[pl.ds(base + k * 16, 16)]
            a = jnp.maximum(a, v) if ismax else jnp.minimum(a, v)
        return _bcast_max(a) if ismax else _bcast_min(a)

    def _rmw(ref, pos, valv):
        cb = (pos // 16) * 16
        lane = pos - cb
        ch = ref[pl.ds(cb, 16)]
        ref[pl.ds(cb, 16)] = jnp.where(iota16 == lane, valv, ch)

    idxg_vec = jnp.full((16,), video * _T, jnp.int32)

    saved = []
    for slot in range(12):
        is_top = slot >= _K
        if slot == _K:
            for tprev, gprev in saved:
                _rmw(srow_a, tprev, gprev)
        sref = srow_a
        bref = bmax if is_top else bmin
        ch0 = bref[pl.ds(0, 16)]
        ch1 = bref[pl.ds(16, 16)]
        ch2 = bref[pl.ds(32, 16)]
        ch3 = bref[pl.ds(48, 16)]
        if is_top:
            gmv = _bcast_max(jnp.maximum(jnp.maximum(ch0, ch1),
                                         jnp.maximum(ch2, ch3)))
        else:
            gmv = _bcast_min(jnp.minimum(jnp.minimum(ch0, ch1),
                                         jnp.minimum(ch2, ch3)))
        jcand = jnp.full((16,), 1 << 30, jnp.int32)
        for q, chq in enumerate((ch0, ch1, ch2, ch3)):
            jcand = jnp.minimum(jcand,
                                jnp.where(chq == gmv, iota16 + q * 16, ibig))
        jstar = _bcast_min(jcand)[0]
        base = jstar * _BLK
        tstar = _find_first(sref, base, gmv)
        maskv = jnp.full((16,), ninf if is_top else pinf, jnp.float32)
        _rmw(sref, tstar, maskv)
        _rmw(bref, jstar, _block_stat(sref, base, is_top))
        if not is_top:
            saved.append((tstar, gmv))
        idxg_vec = jnp.where(iota16 == slot, video * _T + tstar, idxg_vec)

    idxg_r[...] = idxg_vec
    pltpu.async_copy(feat_hbm.at[idxg_r], rows_r, sem).wait()

    cacc = jnp.zeros((16,), jnp.float32)
    for k in range(8):
        cv = cvec_r[pl.ds(k * 16, 16)]
        cacc = cacc + cv * cv
    cn2v = _bcast_sum(cacc)

    dv = jnp.zeros((16,), jnp.float32)
    nv = jnp.ones((16,), jnp.float32)
    for r in range(12):
        ad = jnp.zeros((16,), jnp.float32)
        an = jnp.zeros((16,), jnp.float32)
        for k in range(8):
            v = rows_r[r, pl.ds(k * 16, 16)]
            cv = cvec_r[pl.ds(k * 16, 16)]
            ad = ad + v * cv
            an = an + v * v
        dv = jnp.where(iota16 == r, _bcast_sum(ad), dv)
        nv = jnp.where(iota16 == r, _bcast_sum(an), nv)

    mu = jnp.minimum(_nrsqrt(nv) * _nrsqrt(cn2v), jnp.float32(1.0 / _EPS))
    d = (1.0 - dv * mu) * 0.5
    dn = jnp.take(d, jnp.minimum(iota16 + _K, jnp.int32(15)))
    z = jnp.maximum(d - dn + 1.0, 0.0)
    z = jnp.where(iota16 < _K, z, jnp.float32(0.0))
    pvv = _bcast_sum(z)

    pref[...] = jnp.where(iota16 == 0, pvv, jnp.float32(0.0))
    pltpu.sync_copy(pref, pvrows_hbm.at[wid])
    plsc.subcore_barrier()

    @pl.when(sid == 0)
    def _agg():
        pltpu.sync_copy(pvrows_hbm.at[pl.ds(cid * 16, 16)], tmp_r)
        tot = jnp.zeros((16,), jnp.float32)
        for v2 in range(16):
            tot = tot + tmp_r[v2, pl.ds(0, 16)]
        l2pv = _bcast_sum(tot) * jnp.float32(_BETA / (_HALF * _K))
        pref[...] = jnp.where(iota16 == 0, l2pv, jnp.float32(0.0))
        pltpu.sync_copy(pref, out_hbm.at[cid])


def kernel(feat, score, centers):
    score2 = score.reshape(2 * _HALF, _T)
    feat2 = feat.reshape(2 * _HALF * _T, _FEAT_DIM)
    cen2 = centers.reshape(1, _FEAT_DIM)

    l1 = pl.pallas_call(
        _loss1_body,
        grid=(_HALF,),
        in_specs=[
            pl.BlockSpec((1, _T, _FEAT_DIM), lambda i: (i, 0, 0)),
            pl.BlockSpec((1, _FEAT_DIM), lambda i: (0, 0)),
        ],
        out_specs=pl.BlockSpec((1, 1), lambda i: (0, 0),
                               memory_space=pltpu.SMEM),
        out_shape=jax.ShapeDtypeStruct((1, 1), jnp.float32),
        scratch_shapes=[pltpu.SMEM((1, 1), jnp.float32)],
    )(feat, cen2)

    mesh = plsc.VectorSubcoreMesh(core_axis_name="c", subcore_axis_name="s")
    l2, _ = pl.kernel(
        _sc_triplet,
        out_type=[jax.ShapeDtypeStruct((2, 16), jnp.float32),
                  jax.ShapeDtypeStruct((2 * _HALF // 2, 16), jnp.float32)],
        mesh=mesh,
        scratch_types=[
            pltpu.VMEM((_T,), jnp.float32),
            pltpu.VMEM((_NBLK,), jnp.float32),
            pltpu.VMEM((_NBLK,), jnp.float32),
            pltpu.VMEM((_FEAT_DIM,), jnp.float32),
            pltpu.VMEM((16,), jnp.int32),
            pltpu.VMEM((16, _FEAT_DIM), jnp.float32),
            pltpu.VMEM((16,), jnp.float32),
            pltpu.VMEM((16, 16), jnp.float32),
            pltpu.SemaphoreType.DMA,
        ],
    )(score2, feat2, centers)

    return l1[0, 0] + l2[0, 0] + l2[1, 0]


# speedup vs baseline: 1.4396x; 1.1383x over previous
import jax
import jax.numpy as jnp
from jax import lax
from jax.experimental import pallas as pl
from jax.experimental.pallas import tpu as pltpu
from jax.experimental.pallas import tpu_sc as plsc

_FEAT_DIM = 128
_ALPHA = 0.001
_BETA = 0.5
_EPS = 1e-8
_T = 8192
_HALF = 32
_K = 6
_NBLK = 64
_BLK = 128


def _loss1_body(nor_ref, cen_ref, out_ref, acc_ref):
    i = pl.program_id(0)

    @pl.when(i == 0)
    def _init():
        acc_ref[0, 0] = jnp.float32(0.0)

    x = nor_ref[...].reshape(2 * _T, _FEAT_DIM)
    c = cen_ref[0]
    cinv = lax.rsqrt(jnp.sum(c * c))
    xb = x.astype(jnp.bfloat16)
    xcat = jnp.concatenate([xb, xb * xb], axis=1)
    cb = c.astype(jnp.bfloat16)
    z = jnp.zeros((_FEAT_DIM,), jnp.bfloat16)
    o = jnp.ones((_FEAT_DIM,), jnp.bfloat16)
    w = jnp.stack([jnp.concatenate([cb, z]), jnp.concatenate([z, o])], axis=1)
    p = lax.dot_general(xcat, w, (((1,), (0,)), ((), ())),
                        preferred_element_type=jnp.float32)
    pt = p.T
    dots = pt[0:1, :]
    n2 = pt[1:2, :]
    r = jnp.minimum(lax.rsqrt(n2) * cinv, jnp.float32(1.0 / _EPS))
    acc_ref[0, 0] += jnp.sum(dots * r)

    @pl.when(i == pl.num_programs(0) - 1)
    def _fin():
        mean_cos = acc_ref[0, 0] / jnp.float32(_HALF * _T)
        out_ref[0, 0] = _ALPHA * (1.0 - mean_cos)


def _iota16():
    return lax.broadcasted_iota(jnp.int32, (16,), 0)


def _bcast_max(v):
    it = _iota16()
    for sh in (1, 2, 4, 8):
        v = jnp.maximum(v, jnp.take(v, it ^ sh))
    return v


def _bcast_min(v):
    it = _iota16()
    for sh in (1, 2, 4, 8):
        v = jnp.minimum(v, jnp.take(v, it ^ sh))
    return v


def _bcast_sum(v):
    it = _iota16()
    for sh in (1, 2, 4, 8):
        v = v + jnp.take(v, it ^ sh)
    return v


def _nrsqrt(x):
    xh = x * 0.5
    i = lax.bitcast_convert_type(x, jnp.int32)
    i = jnp.int32(0x5F3759DF) - lax.shift_right_logical(i, 1)
    y = lax.bitcast_convert_type(i, jnp.float32)
    for _ in range(4):
        y = y * (1.5 - xh * y * y)
    return y


def _sc_triplet(score_hbm, feat_hbm, cen_hbm, out_hbm, pvrows_hbm,
                srow_a, bmax, bmin, cvec_r, idxg_r, rows_r,
                pref, tmp_r, sem):
    cid = lax.axis_index("c")
    sid = lax.axis_index("s")
    wid = cid * 16 + sid
    video = _HALF + wid
    iota16 = _iota16()
    ninf = jnp.float32(-jnp.inf)
    pinf = jnp.float32(jnp.inf)
    ibig = jnp.int32(1 << 30)

    pltpu.sync_copy(score_hbm.at[video], srow_a)
    pltpu.sync_copy(cen_hbm, cvec_r)

    def _blk(j, carry):
        base = j * _BLK
        mx = srow_a[pl.ds(base, 16)]
        mn = mx
        for k in range(1, 8):
            v = srow_a[pl.ds(base + k * 16, 16)]
            mx = jnp.maximum(mx, v)
            mn = jnp.minimum(mn, v)
        bmxv = _bcast_max(mx)
        bmnv = _bcast_min(mn)
        cb = (j // 16) * 16
        lane = j - cb
        bmax[pl.ds(cb, 16)] = jnp.where(iota16 == lane, bmxv, bmax[pl.ds(cb, 16)])
        bmin[pl.ds(cb, 16)] = jnp.where(iota16 == lane, bmnv, bmin[pl.ds(cb, 16)])
        return carry

    lax.fori_loop(0, _NBLK, _blk, 0, unroll=4)

    def _find_first(ref, base, valv):
        t = jnp.full((16,), 1 << 30, jnp.int32)
        for k in range(8):
            v = ref[pl.ds(base + k * 16, 16)]
            cand = jnp.where(v == valv, iota16 + (base + k * 16), ibig)
            t = jnp.minimum(t, cand)
        return _bcast_min(t)[0]

    def _block_stat(ref, base, ismax):
        a = ref[pl.ds(base, 16)]
        for k in range(1, 8):
            v = ref[pl.ds(base + k * 16, 16)]
            a = jnp.maximum(a, v) if ismax else jnp.minimum(a, v)
        return _bcast_max(a) if ismax else _bcast_min(a)

    def _rmw(ref, pos, valv):
        cb = (pos // 16) * 16
        lane = pos - cb
        ch = ref[pl.ds(cb, 16)]
        ref[pl.ds(cb, 16)] = jnp.where(iota16 == lane, valv, ch)

    idxg_vec = jnp.full((16,), video * _T, jnp.int32)

    saved = []
    for slot in range(12):
        is_top = slot >= _K
        if slot == _K:
            for tprev, gprev in saved:
                _rmw(srow_a, tprev, gprev)
        sref = srow_a
        bref = bmax if is_top else bmin
        ch0 = bref[pl.ds(0, 16)]
        ch1 = bref[pl.ds(16, 16)]
        ch2 = bref[pl.ds(32, 16)]
        ch3 = bref[pl.ds(48, 16)]
        if is_top:
            gmv = _bcast_max(jnp.maximum(jnp.maximum(ch0, ch1),
                                         jnp.maximum(ch2, ch3)))
        else:
            gmv = _bcast_min(jnp.minimum(jnp.minimum(ch0, ch1),
                                         jnp.minimum(ch2, ch3)))
        jcand = jnp.full((16,), 1 << 30, jnp.int32)
        for q, chq in enumerate((ch0, ch1, ch2, ch3)):
            jcand = jnp.minimum(jcand,
                                jnp.where(chq == gmv, iota16 + q * 16, ibig))
        jstar = _bcast_min(jcand)[0]
        base = jstar * _BLK
        tstar = _find_first(sref, base, gmv)
        maskv = jnp.full((16,), ninf if is_top else pinf, jnp.float32)
        _rmw(sref, tstar, maskv)
        _rmw(bref, jstar, _block_stat(sref, base, is_top))
        if not is_top:
            saved.append((tstar, gmv))
        idxg_vec = jnp.where(iota16 == slot, video * _T + tstar, idxg_vec)

    idxg_r[...] = idxg_vec
    pltpu.async_copy(feat_hbm.at[idxg_r], rows_r, sem).wait()

    cacc = jnp.zeros((16,), jnp.float32)
    for k in range(8):
        cv = cvec_r[pl.ds(k * 16, 16)]
        cacc = cacc + cv * cv
    cn2v = _bcast_sum(cacc)

    dv = jnp.zeros((16,), jnp.float32)
    nv = jnp.ones((16,), jnp.float32)
    for r in range(12):
        ad = jnp.zeros((16,), jnp.float32)
        an = jnp.zeros((16,), jnp.float32)
        for k in range(8):
            v = rows_r[r, pl.ds(k * 16, 16)]
            cv = cvec_r[pl.ds(k * 16, 16)]
            ad = ad + v * cv
            an = an + v * v
        dv = jnp.where(iota16 == r, _bcast_sum(ad), dv)
        nv = jnp.where(iota16 == r, _bcast_sum(an), nv)

    mu = jnp.minimum(_nrsqrt(nv) * _nrsqrt(cn2v), jnp.float32(1.0 / _EPS))
    d = (1.0 - dv * mu) * 0.5
    dn = jnp.take(d, jnp.minimum(iota16 + _K, jnp.int32(15)))
    z = jnp.maximum(d - dn + 1.0, 0.0)
    z = jnp.where(iota16 < _K, z, jnp.float32(0.0))
    pvv = _bcast_sum(z)

    pref[...] = jnp.where(iota16 == 0, pvv, jnp.float32(0.0))
    pltpu.sync_copy(pref, pvrows_hbm.at[wid])
    plsc.subcore_barrier()

    @pl.when(sid == 0)
    def _agg():
        pltpu.sync_copy(pvrows_hbm.at[pl.ds(cid * 16, 16)], tmp_r)
        tot = jnp.zeros((16,), jnp.float32)
        for v2 in range(16):
            tot = tot + tmp_r[v2, pl.ds(0, 16)]
        l2pv = _bcast_sum(tot) * jnp.float32(_BETA / (_HALF * _K))
        pref[...] = jnp.where(iota16 == 0, l2pv, jnp.float32(0.0))
        pltpu.sync_copy(pref, out_hbm.at[cid])


def kernel(feat, score, centers):
    score2 = score.reshape(2 * _HALF, _T)
    feat2 = feat.reshape(2 * _HALF * _T, _FEAT_DIM)
    cen2 = centers.reshape(1, _FEAT_DIM)

    l1 = pl.pallas_call(
        _loss1_body,
        grid=(_HALF // 2,),
        in_specs=[
            pl.BlockSpec((2, _T, _FEAT_DIM), lambda i: (i, 0, 0)),
            pl.BlockSpec((1, _FEAT_DIM), lambda i: (0, 0)),
        ],
        out_specs=pl.BlockSpec((1, 1), lambda i: (0, 0),
                               memory_space=pltpu.SMEM),
        out_shape=jax.ShapeDtypeStruct((1, 1), jnp.float32),
        scratch_shapes=[pltpu.SMEM((1, 1), jnp.float32)],
    )(feat, cen2)

    mesh = plsc.VectorSubcoreMesh(core_axis_name="c", subcore_axis_name="s")
    l2, _ = pl.kernel(
        _sc_triplet,
        out_type=[jax.ShapeDtypeStruct((2, 16), jnp.float32),
                  jax.ShapeDtypeStruct((2 * _HALF // 2, 16), jnp.float32)],
        mesh=mesh,
        scratch_types=[
            pltpu.VMEM((_T,), jnp.float32),
            pltpu.VMEM((_NBLK,), jnp.float32),
            pltpu.VMEM((_NBLK,), jnp.float32),
            pltpu.VMEM((_FEAT_DIM,), jnp.float32),
            pltpu.VMEM((16,), jnp.int32),
            pltpu.VMEM((16, _FEAT_DIM), jnp.float32),
            pltpu.VMEM((16,), jnp.float32),
            pltpu.VMEM((16, 16), jnp.float32),
            pltpu.SemaphoreType.DMA,
        ],
    )(score2, feat2, centers)

    return l1[0, 0] + l2[0, 0] + l2[1, 0]
